# baseline (speedup 1.0000x reference)
import functools

import jax
import jax.numpy as jnp
from jax import lax
from jax.experimental import pallas as pl
from jax.experimental.pallas import tpu as pltpu
from jax.experimental.pallas import tpu_sc as plsc

N = 10000
D = 256
HID = 8
HEADS = 8
NC = 40
F1 = HEADS * HID
NPAD = 10240
E_TOT = 160000 + N
NWORK = 32
CHUNK = 128
NCHUNK = -(-E_TOT // (NWORK * CHUNK))
E_PAD = NWORK * NCHUNK * CHUNK
ZR = NPAD // 16
LANES = 16
BR = 2048


def _mesh():
    return plsc.VectorSubcoreMesh(core_axis_name="c", subcore_axis_name="s",
                                  num_cores=2, num_subcores=16)


def _make_att(A):
    vshape = (CHUNK, A) if A > 1 else (CHUNK,)
    nshape = (NPAD, A) if A > 1 else (NPAD,)
    NV = CHUNK * A // LANES

    def body(src_h, dst_h, as_h, ad_h, zer_h, den_h, ex_h,
             idx_s, idx_d, asg, adg, exv, den_sh, sem0, sem1, sem2, sem3):
        cid = lax.axis_index("c")
        sid = lax.axis_index("s")
        wid = sid * 2 + cid
        pltpu.sync_copy(zer_h.at[pl.ds(sid * ZR, ZR)],
                        den_sh.at[pl.ds(sid * ZR, ZR)])
        plsc.subcore_barrier()
        pltpu.sync_copy(src_h.at[wid], idx_s)
        pltpu.sync_copy(dst_h.at[wid], idx_d)
        lanes = lax.iota(jnp.int32, LANES)

        def chunk_body(j, carry):
            cps = pltpu.async_copy(as_h.at[idx_s.at[j]], asg, sem0)
            cpd = pltpu.async_copy(ad_h.at[idx_d.at[j]], adg, sem1)
            cps.wait()
            cpd.wait()

            rowoff = lanes // A if A > 1 else lanes
            colv = lanes % A
            rpv = LANES // A if A > 1 else LANES

            def vec_body(v, c2):
                row = v * rpv + rowoff
                idxs = [row, colv] if A > 1 else [row]
                a = plsc.load_gather(asg, idxs) + plsc.load_gather(adg, idxs)
                a = jnp.where(a > 0, a, 0.2 * a)
                plsc.store_scatter(exv, idxs, jnp.exp(a))
                return c2

            lax.fori_loop(0, NV, vec_body, 0, unroll=4)
            cp1 = pltpu.async_copy(exv, ex_h.at[wid, j], sem2)
            cp2 = pltpu.async_copy(exv, den_sh.at[idx_d.at[j]], sem3, add=True)
            cp1.wait()
            cp2.wait()
            return carry

        lax.fori_loop(0, NCHUNK, chunk_body, 0)
        plsc.subcore_barrier()
        pltpu.sync_copy(den_sh.at[pl.ds(sid * ZR, ZR)],
                        den_h.at[cid].at[pl.ds(sid * ZR, ZR)])

    @jax.jit
    def run(src3, dst3, a_s, a_d, zer):
        return pl.kernel(
            body,
            out_type=[
                jax.ShapeDtypeStruct((2,) + nshape, jnp.float32),
                jax.ShapeDtypeStruct((NWORK, NCHUNK) + vshape, jnp.float32),
            ],
            mesh=_mesh(),
            compiler_params=pltpu.CompilerParams(needs_layout_passes=False,
                                                 use_tc_tiling_on_sc=False),
            scratch_types=[
                pltpu.VMEM((NCHUNK, CHUNK), jnp.int32),
                pltpu.VMEM((NCHUNK, CHUNK), jnp.int32),
                pltpu.VMEM(vshape, jnp.float32),
                pltpu.VMEM(vshape, jnp.float32),
                pltpu.VMEM(vshape, jnp.float32),
                pltpu.VMEM_SHARED(nshape, jnp.float32),
                pltpu.SemaphoreType.DMA,
                pltpu.SemaphoreType.DMA,
                pltpu.SemaphoreType.DMA,
                pltpu.SemaphoreType.DMA,
            ],
        )(src3, dst3, a_s, a_d, zer)

    return run


def _make_agg(F, A):
    vshape = (CHUNK, A) if A > 1 else (CHUNK,)
    nshape = (NPAD, A) if A > 1 else (NPAD,)
    NVC = CHUNK * A // LANES
    NVF = CHUNK * F // LANES
    FH = F // A
    import math
    LCM = math.lcm(LANES, F)
    UNF = LCM // LANES
    RPO = LCM // F

    def body(src_h, dst_h, ex_h, d0_h, d1_h, h_h, zer_h, out_h,
             idx_s, idx_d, exv, dg0, dg1, coefv, hg, wv, out_sh,
             sem0, sem1, sem2, sem3):
        cid = lax.axis_index("c")
        sid = lax.axis_index("s")
        wid = sid * 2 + cid
        pltpu.sync_copy(zer_h.at[pl.ds(sid * ZR, ZR)],
                        out_sh.at[pl.ds(sid * ZR, ZR)])
        plsc.subcore_barrier()
        pltpu.sync_copy(src_h.at[wid], idx_s)
        pltpu.sync_copy(dst_h.at[wid], idx_d)
        lanes = lax.iota(jnp.int32, LANES)
        wcol = [(LANES * k + lanes) % F for k in range(UNF)]
        wroff = [(LANES * k + lanes) // F for k in range(UNF)]
        whead = [c // FH for c in wcol]

        def chunk_body(j, carry):
            cp0 = pltpu.async_copy(d0_h.at[idx_d.at[j]], dg0, sem0)
            cp1 = pltpu.async_copy(d1_h.at[idx_d.at[j]], dg1, sem1)
            cph = pltpu.async_copy(h_h.at[idx_s.at[j]], hg, sem2)
            pltpu.sync_copy(ex_h.at[wid, j], exv)
            cp0.wait()
            cp1.wait()

            rowoff = lanes // A if A > 1 else lanes
            colv = lanes % A
            rpv = LANES // A if A > 1 else LANES

            def cvec(v, c2):
                row = v * rpv + rowoff
                idxs = [row, colv] if A > 1 else [row]
                e = plsc.load_gather(exv, idxs)
                d = plsc.load_gather(dg0, idxs) + plsc.load_gather(dg1, idxs)
                plsc.store_scatter(coefv, idxs, e / (d + 1e-16))
                return c2

            lax.fori_loop(0, NVC, cvec, 0, unroll=4)
            cph.wait()

            def wvec(vv, c2):
                base_row = vv * RPO
                for k in range(UNF):
                    row = base_row + wroff[k]
                    hv = plsc.load_gather(hg, [row, wcol[k]])
                    if A > 1:
                        cv = plsc.load_gather(coefv, [row, whead[k]])
                    else:
                        cv = plsc.load_gather(coefv, [row])
                    plsc.store_scatter(wv, [row, wcol[k]], hv * cv)
                return c2

            lax.fori_loop(0, NVF // UNF, wvec, 0, unroll=2)
            pltpu.async_copy(wv, out_sh.at[idx_d.at[j]], sem3, add=True).wait()
            return carry

        lax.fori_loop(0, NCHUNK, chunk_body, 0)
        plsc.subcore_barrier()
        pltpu.sync_copy(out_sh.at[pl.ds(sid * ZR, ZR)],
                        out_h.at[cid].at[pl.ds(sid * ZR, ZR)])

    @jax.jit
    def run(src3, dst3, ex, d0, d1, h, zer):
        return pl.kernel(
            body,
            out_type=jax.ShapeDtypeStruct((2, NPAD, F), jnp.float32),
            mesh=_mesh(),
            compiler_params=pltpu.CompilerParams(needs_layout_passes=False,
                                                 use_tc_tiling_on_sc=False),
            scratch_types=[
                pltpu.VMEM((NCHUNK, CHUNK), jnp.int32),
                pltpu.VMEM((NCHUNK, CHUNK), jnp.int32),
                pltpu.VMEM(vshape, jnp.float32),
                pltpu.VMEM(vshape, jnp.float32),
                pltpu.VMEM(vshape, jnp.float32),
                pltpu.VMEM(vshape, jnp.float32),
                pltpu.VMEM((CHUNK, F), jnp.float32),
                pltpu.VMEM((CHUNK, F), jnp.float32),
                pltpu.VMEM_SHARED((NPAD, F), jnp.float32),
                pltpu.SemaphoreType.DMA,
                pltpu.SemaphoreType.DMA,
                pltpu.SemaphoreType.DMA,
                pltpu.SemaphoreType.DMA,
            ],
        )(src3, dst3, ex, d0, d1, h, zer)

    return run


def _proj1_body(x_ref, w_ref, bs_ref, bd_ref, h_ref, as_ref, ad_ref):
    h = jnp.dot(x_ref[...], w_ref[...], preferred_element_type=jnp.float32)
    h_ref[...] = h
    as_ref[...] = jnp.dot(h, bs_ref[...], preferred_element_type=jnp.float32)
    ad_ref[...] = jnp.dot(h, bd_ref[...], preferred_element_type=jnp.float32)


@jax.jit
def _tc_proj1(x_p, W1, BDs, BDd):
    grid = (NPAD // BR,)
    return pl.pallas_call(
        _proj1_body,
        grid=grid,
        in_specs=[
            pl.BlockSpec((BR, D), lambda i: (i, 0)),
            pl.BlockSpec((D, F1), lambda i: (0, 0)),
            pl.BlockSpec((F1, HEADS), lambda i: (0, 0)),
            pl.BlockSpec((F1, HEADS), lambda i: (0, 0)),
        ],
        out_specs=[
            pl.BlockSpec((BR, F1), lambda i: (i, 0)),
            pl.BlockSpec((BR, HEADS), lambda i: (i, 0)),
            pl.BlockSpec((BR, HEADS), lambda i: (i, 0)),
        ],
        out_shape=[
            jax.ShapeDtypeStruct((NPAD, F1), jnp.float32),
            jax.ShapeDtypeStruct((NPAD, HEADS), jnp.float32),
            jax.ShapeDtypeStruct((NPAD, HEADS), jnp.float32),
        ],
    )(x_p, W1, BDs, BDd)


def _mid_body(p_ref, b1_ref, w2_ref, as2_ref, ad2_ref, h2_ref, a2s_ref, a2d_ref):
    t = p_ref[0] + p_ref[1] + b1_ref[...]
    t = jnp.where(t > 0, t, jnp.exp(jnp.minimum(t, 0.0)) - 1.0)
    h2 = jnp.dot(t, w2_ref[...], preferred_element_type=jnp.float32)
    h2_ref[...] = h2
    a2s_ref[...] = jnp.dot(h2, as2_ref[...], preferred_element_type=jnp.float32)
    a2d_ref[...] = jnp.dot(h2, ad2_ref[...], preferred_element_type=jnp.float32)


@jax.jit
def _tc_mid(out1p, b1r, W2, as2w, ad2w):
    grid = (NPAD // BR,)
    return pl.pallas_call(
        _mid_body,
        grid=grid,
        in_specs=[
            pl.BlockSpec((2, BR, F1), lambda i: (0, i, 0)),
            pl.BlockSpec((1, F1), lambda i: (0, 0)),
            pl.BlockSpec((F1, NC), lambda i: (0, 0)),
            pl.BlockSpec((NC, 1), lambda i: (0, 0)),
            pl.BlockSpec((NC, 1), lambda i: (0, 0)),
        ],
        out_specs=[
            pl.BlockSpec((BR, NC), lambda i: (i, 0)),
            pl.BlockSpec((BR, 1), lambda i: (i, 0)),
            pl.BlockSpec((BR, 1), lambda i: (i, 0)),
        ],
        out_shape=[
            jax.ShapeDtypeStruct((NPAD, NC), jnp.float32),
            jax.ShapeDtypeStruct((NPAD, 1), jnp.float32),
            jax.ShapeDtypeStruct((NPAD, 1), jnp.float32),
        ],
    )(out1p, b1r, W2, as2w, ad2w)


def _final_body(p_ref, b2_ref, o_ref):
    o_ref[...] = p_ref[0] + p_ref[1] + b2_ref[...]


@jax.jit
def _tc_final(out2p, b2r):
    grid = (NPAD // BR,)
    return pl.pallas_call(
        _final_body,
        grid=grid,
        in_specs=[
            pl.BlockSpec((2, BR, NC), lambda i: (0, i, 0)),
            pl.BlockSpec((1, NC), lambda i: (0, 0)),
        ],
        out_specs=pl.BlockSpec((BR, NC), lambda i: (i, 0)),
        out_shape=jax.ShapeDtypeStruct((NPAD, NC), jnp.float32),
    )(out2p, b2r)


_att8 = _make_att(HEADS)
_att1 = _make_att(1)
_agg64 = _make_agg(F1, HEADS)
_agg40 = _make_agg(NC, 1)


def kernel(x, edge_index, W1, att_src1, att_dst1, b1, W2, att_src2, att_dst2, b2):
    ei = edge_index.astype(jnp.int32)
    loops = jnp.arange(N, dtype=jnp.int32)
    pad = jnp.full((E_PAD - E_TOT,), N, jnp.int32)
    src3 = jnp.concatenate([ei[0], loops, pad]).reshape(NWORK, NCHUNK, CHUNK)
    dst3 = jnp.concatenate([ei[1], loops, pad]).reshape(NWORK, NCHUNK, CHUNK)
    x_p = jnp.zeros((NPAD, D), jnp.float32).at[:N].set(x)
    eye = jnp.eye(HEADS, dtype=jnp.float32)
    BDs = (eye[:, None, :] * att_src1[:, :, None]).reshape(F1, HEADS)
    BDd = (eye[:, None, :] * att_dst1[:, :, None]).reshape(F1, HEADS)

    h1, a1s, a1d = _tc_proj1(x_p, W1, BDs, BDd)

    zeros8 = jnp.zeros((NPAD, HEADS), jnp.float32)
    den1p, ex1 = _att8(src3, dst3, a1s, a1d, zeros8)

    zeros64 = jnp.zeros((NPAD, F1), jnp.float32)
    out1p = _agg64(src3, dst3, ex1, den1p[0], den1p[1], h1, zeros64)

    h2, a2s, a2d = _tc_mid(out1p, b1.reshape(1, F1), W2,
                           att_src2.reshape(NC, 1), att_dst2.reshape(NC, 1))
    a2s = a2s.reshape(NPAD)
    a2d = a2d.reshape(NPAD)

    zeros1 = jnp.zeros((NPAD,), jnp.float32)
    den2p, ex2 = _att1(src3, dst3, a2s, a2d, zeros1)

    zeros40 = jnp.zeros((NPAD, NC), jnp.float32)
    out2p = _agg40(src3, dst3, ex2, den2p[0], den2p[1], h2, zeros40)

    out = _tc_final(out2p, b2.reshape(1, NC))
    return out[:N]


# speedup vs baseline: 1.4297x; 1.4297x over previous
import functools

import jax
import jax.numpy as jnp
from jax import lax
from jax.experimental import pallas as pl
from jax.experimental.pallas import tpu as pltpu
from jax.experimental.pallas import tpu_sc as plsc

N = 10000
D = 256
HID = 8
HEADS = 8
NC = 40
F1 = HEADS * HID
NPAD = 10240
E_TOT = 160000 + N
NWORK = 32
CHUNK = 128
NCHUNK = -(-E_TOT // (NWORK * CHUNK))
E_PAD = NWORK * NCHUNK * CHUNK
ZR = NPAD // 16
LANES = 16
BR = 2048


def _mesh():
    return plsc.VectorSubcoreMesh(core_axis_name="c", subcore_axis_name="s",
                                  num_cores=2, num_subcores=16)


def _make_att(A):
    vshape = (CHUNK, A) if A > 1 else (CHUNK,)
    nshape = (NPAD, A) if A > 1 else (NPAD,)
    NV = CHUNK * A // LANES

    def body(src_h, dst_h, as_h, ad_h, zer_h, den_h, ex_h,
             idx_s, idx_d, asg0, asg1, adg0, adg1, exv0, exv1, den_sh,
             semg0, semg1, seme0, seme1, semc0, semc1):
        cid = lax.axis_index("c")
        sid = lax.axis_index("s")
        wid = sid * 2 + cid
        asgs, adgs, exvs = (asg0, asg1), (adg0, adg1), (exv0, exv1)
        semg, seme, semc = (semg0, semg1), (seme0, seme1), (semc0, semc1)
        pltpu.sync_copy(zer_h.at[pl.ds(sid * ZR, ZR)],
                        den_sh.at[pl.ds(sid * ZR, ZR)])
        plsc.subcore_barrier()
        pltpu.sync_copy(src_h.at[wid], idx_s)
        pltpu.sync_copy(dst_h.at[wid], idx_d)
        lanes = lax.iota(jnp.int32, LANES)
        rowoff = lanes // A if A > 1 else lanes
        colv = lanes % A
        rpv = LANES // A if A > 1 else LANES

        def issue(j, b):
            pltpu.async_copy(as_h.at[idx_s.at[j]], asgs[b], semg[b])
            pltpu.async_copy(ad_h.at[idx_d.at[j]], adgs[b], semg[b])

        def drain(j, b):
            pltpu.make_async_copy(as_h.at[idx_s.at[j]], asgs[b], semg[b]).wait()
            pltpu.make_async_copy(ad_h.at[idx_d.at[j]], adgs[b], semg[b]).wait()

        def drain_out(j, b):
            pltpu.make_async_copy(exvs[b], ex_h.at[wid, j], seme[b]).wait()
            pltpu.make_async_copy(exvs[b], den_sh.at[idx_d.at[j]], semc[b]).wait()

        issue(0, 0)

        def pair_body(jj, carry):
            for b in range(2):
                j = 2 * jj + b
                nb = 1 - b

                @pl.when(j + 1 < NCHUNK)
                def _():
                    issue(j + 1, nb)

                drain(j, b)

                @pl.when(j >= 2)
                def _():
                    drain_out(j - 2, b)

                def vec_body(v, c2):
                    row = v * rpv + rowoff
                    idxs = [row, colv] if A > 1 else [row]
                    a = (plsc.load_gather(asgs[b], idxs)
                         + plsc.load_gather(adgs[b], idxs))
                    a = jnp.where(a > 0, a, 0.2 * a)
                    plsc.store_scatter(exvs[b], idxs, jnp.exp(a))
                    return c2

                lax.fori_loop(0, NV, vec_body, 0, unroll=4)
                pltpu.async_copy(exvs[b], ex_h.at[wid, j], seme[b])
                pltpu.async_copy(exvs[b], den_sh.at[idx_d.at[j]], semc[b],
                                 add=True)
            return carry

        lax.fori_loop(0, NCHUNK // 2, pair_body, 0)
        drain_out(NCHUNK - 2, 0)
        drain_out(NCHUNK - 1, 1)
        plsc.subcore_barrier()
        pltpu.sync_copy(den_sh.at[pl.ds(sid * ZR, ZR)],
                        den_h.at[cid].at[pl.ds(sid * ZR, ZR)])

    @jax.jit
    def run(src3, dst3, a_s, a_d, zer):
        return pl.kernel(
            body,
            out_type=[
                jax.ShapeDtypeStruct((2,) + nshape, jnp.float32),
                jax.ShapeDtypeStruct((NWORK, NCHUNK) + vshape, jnp.float32),
            ],
            mesh=_mesh(),
            compiler_params=pltpu.CompilerParams(needs_layout_passes=False,
                                                 use_tc_tiling_on_sc=False),
            scratch_types=(
                [pltpu.VMEM((NCHUNK, CHUNK), jnp.int32)] * 2
                + [pltpu.VMEM(vshape, jnp.float32)] * 6
                + [pltpu.VMEM_SHARED(nshape, jnp.float32)]
                + [pltpu.SemaphoreType.DMA] * 6
            ),
        )(src3, dst3, a_s, a_d, zer)

    return run


def _make_agg(F, A):
    vshape = (CHUNK, A) if A > 1 else (CHUNK,)
    nshape = (NPAD, A) if A > 1 else (NPAD,)
    NVC = CHUNK * A // LANES
    NVF = CHUNK * F // LANES
    FH = F // A
    import math
    LCM = math.lcm(LANES, F)
    UNF = LCM // LANES
    RPO = LCM // F

    def body(src_h, dst_h, ex_h, d0_h, d1_h, h_h, zer_h, out_h,
             idx_s, idx_d, exv0, exv1, dg00, dg01, dg10, dg11, coefv,
             hg0, hg1, wv0, wv1, out_sh, semg0, semg1, semc0, semc1):
        cid = lax.axis_index("c")
        sid = lax.axis_index("s")
        wid = sid * 2 + cid
        exvs, dg0s, dg1s = (exv0, exv1), (dg00, dg01), (dg10, dg11)
        hgs, wvs = (hg0, hg1), (wv0, wv1)
        semg, semc = (semg0, semg1), (semc0, semc1)
        pltpu.sync_copy(zer_h.at[pl.ds(sid * ZR, ZR)],
                        out_sh.at[pl.ds(sid * ZR, ZR)])
        plsc.subcore_barrier()
        pltpu.sync_copy(src_h.at[wid], idx_s)
        pltpu.sync_copy(dst_h.at[wid], idx_d)
        lanes = lax.iota(jnp.int32, LANES)
        wcol = [(LANES * k + lanes) % F for k in range(UNF)]
        wroff = [(LANES * k + lanes) // F for k in range(UNF)]
        whead = [c // FH for c in wcol]
        rowoff = lanes // A if A > 1 else lanes
        colv = lanes % A
        rpv = LANES // A if A > 1 else LANES

        def issue(j, b):
            pltpu.async_copy(d0_h.at[idx_d.at[j]], dg0s[b], semg[b])
            pltpu.async_copy(d1_h.at[idx_d.at[j]], dg1s[b], semg[b])
            pltpu.async_copy(ex_h.at[wid, j], exvs[b], semg[b])
            pltpu.async_copy(h_h.at[idx_s.at[j]], hgs[b], semg[b])

        def drain(j, b):
            pltpu.make_async_copy(d0_h.at[idx_d.at[j]], dg0s[b], semg[b]).wait()
            pltpu.make_async_copy(d1_h.at[idx_d.at[j]], dg1s[b], semg[b]).wait()
            pltpu.make_async_copy(ex_h.at[wid, j], exvs[b], semg[b]).wait()
            pltpu.make_async_copy(h_h.at[idx_s.at[j]], hgs[b], semg[b]).wait()

        def drain_out(j, b):
            pltpu.make_async_copy(wvs[b], out_sh.at[idx_d.at[j]],
                                  semc[b]).wait()

        issue(0, 0)

        def pair_body(jj, carry):
            for b in range(2):
                j = 2 * jj + b
                nb = 1 - b

                @pl.when(j + 1 < NCHUNK)
                def _():
                    issue(j + 1, nb)

                drain(j, b)

                @pl.when(j >= 2)
                def _():
                    drain_out(j - 2, b)

                def cvec(v, c2):
                    row = v * rpv + rowoff
                    idxs = [row, colv] if A > 1 else [row]
                    e = plsc.load_gather(exvs[b], idxs)
                    d = (plsc.load_gather(dg0s[b], idxs)
                         + plsc.load_gather(dg1s[b], idxs))
                    plsc.store_scatter(coefv, idxs, e / (d + 1e-16))
                    return c2

                lax.fori_loop(0, NVC, cvec, 0, unroll=4)

                def wvec(vv, c2):
                    base_row = vv * RPO
                    for k in range(UNF):
                        row = base_row + wroff[k]
                        hv = plsc.load_gather(hgs[b], [row, wcol[k]])
                        if A > 1:
                            cv = plsc.load_gather(coefv, [row, whead[k]])
                        else:
                            cv = plsc.load_gather(coefv, [row])
                        plsc.store_scatter(wvs[b], [row, wcol[k]], hv * cv)
                    return c2

                lax.fori_loop(0, NVF // UNF, wvec, 0, unroll=2)
                pltpu.async_copy(wvs[b], out_sh.at[idx_d.at[j]], semc[b],
                                 add=True)
            return carry

        lax.fori_loop(0, NCHUNK // 2, pair_body, 0)
        drain_out(NCHUNK - 2, 0)
        drain_out(NCHUNK - 1, 1)
        plsc.subcore_barrier()
        pltpu.sync_copy(out_sh.at[pl.ds(sid * ZR, ZR)],
                        out_h.at[cid].at[pl.ds(sid * ZR, ZR)])

    @jax.jit
    def run(src3, dst3, ex, d0, d1, h, zer):
        return pl.kernel(
            body,
            out_type=jax.ShapeDtypeStruct((2, NPAD, F), jnp.float32),
            mesh=_mesh(),
            compiler_params=pltpu.CompilerParams(needs_layout_passes=False,
                                                 use_tc_tiling_on_sc=False),
            scratch_types=(
                [pltpu.VMEM((NCHUNK, CHUNK), jnp.int32)] * 2
                + [pltpu.VMEM(vshape, jnp.float32)] * 7
                + [pltpu.VMEM((CHUNK, F), jnp.float32)] * 4
                + [pltpu.VMEM_SHARED((NPAD, F), jnp.float32)]
                + [pltpu.SemaphoreType.DMA] * 4
            ),
        )(src3, dst3, ex, d0, d1, h, zer)

    return run


def _proj1_body(x_ref, w_ref, bs_ref, bd_ref, h_ref, as_ref, ad_ref):
    h = jnp.dot(x_ref[...], w_ref[...], preferred_element_type=jnp.float32)
    h_ref[...] = h
    as_ref[...] = jnp.dot(h, bs_ref[...], preferred_element_type=jnp.float32)
    ad_ref[...] = jnp.dot(h, bd_ref[...], preferred_element_type=jnp.float32)


@jax.jit
def _tc_proj1(x_p, W1, BDs, BDd):
    grid = (NPAD // BR,)
    return pl.pallas_call(
        _proj1_body,
        grid=grid,
        in_specs=[
            pl.BlockSpec((BR, D), lambda i: (i, 0)),
            pl.BlockSpec((D, F1), lambda i: (0, 0)),
            pl.BlockSpec((F1, HEADS), lambda i: (0, 0)),
            pl.BlockSpec((F1, HEADS), lambda i: (0, 0)),
        ],
        out_specs=[
            pl.BlockSpec((BR, F1), lambda i: (i, 0)),
            pl.BlockSpec((BR, HEADS), lambda i: (i, 0)),
            pl.BlockSpec((BR, HEADS), lambda i: (i, 0)),
        ],
        out_shape=[
            jax.ShapeDtypeStruct((NPAD, F1), jnp.float32),
            jax.ShapeDtypeStruct((NPAD, HEADS), jnp.float32),
            jax.ShapeDtypeStruct((NPAD, HEADS), jnp.float32),
        ],
    )(x_p, W1, BDs, BDd)


def _mid_body(p_ref, b1_ref, w2_ref, as2_ref, ad2_ref, h2_ref, a2s_ref, a2d_ref):
    t = p_ref[0] + p_ref[1] + b1_ref[...]
    t = jnp.where(t > 0, t, jnp.exp(jnp.minimum(t, 0.0)) - 1.0)
    h2 = jnp.dot(t, w2_ref[...], preferred_element_type=jnp.float32)
    h2_ref[...] = h2
    a2s_ref[...] = jnp.dot(h2, as2_ref[...], preferred_element_type=jnp.float32)
    a2d_ref[...] = jnp.dot(h2, ad2_ref[...], preferred_element_type=jnp.float32)


@jax.jit
def _tc_mid(out1p, b1r, W2, as2w, ad2w):
    grid = (NPAD // BR,)
    return pl.pallas_call(
        _mid_body,
        grid=grid,
        in_specs=[
            pl.BlockSpec((2, BR, F1), lambda i: (0, i, 0)),
            pl.BlockSpec((1, F1), lambda i: (0, 0)),
            pl.BlockSpec((F1, NC), lambda i: (0, 0)),
            pl.BlockSpec((NC, 1), lambda i: (0, 0)),
            pl.BlockSpec((NC, 1), lambda i: (0, 0)),
        ],
        out_specs=[
            pl.BlockSpec((BR, NC), lambda i: (i, 0)),
            pl.BlockSpec((BR, 1), lambda i: (i, 0)),
            pl.BlockSpec((BR, 1), lambda i: (i, 0)),
        ],
        out_shape=[
            jax.ShapeDtypeStruct((NPAD, NC), jnp.float32),
            jax.ShapeDtypeStruct((NPAD, 1), jnp.float32),
            jax.ShapeDtypeStruct((NPAD, 1), jnp.float32),
        ],
    )(out1p, b1r, W2, as2w, ad2w)


def _final_body(p_ref, b2_ref, o_ref):
    o_ref[...] = p_ref[0] + p_ref[1] + b2_ref[...]


@jax.jit
def _tc_final(out2p, b2r):
    grid = (NPAD // BR,)
    return pl.pallas_call(
        _final_body,
        grid=grid,
        in_specs=[
            pl.BlockSpec((2, BR, NC), lambda i: (0, i, 0)),
            pl.BlockSpec((1, NC), lambda i: (0, 0)),
        ],
        out_specs=pl.BlockSpec((BR, NC), lambda i: (i, 0)),
        out_shape=jax.ShapeDtypeStruct((NPAD, NC), jnp.float32),
    )(out2p, b2r)


_att8 = _make_att(HEADS)
_att1 = _make_att(1)
_agg64 = _make_agg(F1, HEADS)
_agg40 = _make_agg(NC, 1)


def kernel(x, edge_index, W1, att_src1, att_dst1, b1, W2, att_src2, att_dst2, b2):
    ei = edge_index.astype(jnp.int32)
    loops = jnp.arange(N, dtype=jnp.int32)
    pad = jnp.full((E_PAD - E_TOT,), N, jnp.int32)
    src3 = jnp.concatenate([ei[0], loops, pad]).reshape(NWORK, NCHUNK, CHUNK)
    dst3 = jnp.concatenate([ei[1], loops, pad]).reshape(NWORK, NCHUNK, CHUNK)
    x_p = jnp.zeros((NPAD, D), jnp.float32).at[:N].set(x)
    eye = jnp.eye(HEADS, dtype=jnp.float32)
    BDs = (eye[:, None, :] * att_src1[:, :, None]).reshape(F1, HEADS)
    BDd = (eye[:, None, :] * att_dst1[:, :, None]).reshape(F1, HEADS)

    h1, a1s, a1d = _tc_proj1(x_p, W1, BDs, BDd)

    zeros8 = jnp.zeros((NPAD, HEADS), jnp.float32)
    den1p, ex1 = _att8(src3, dst3, a1s, a1d, zeros8)

    zeros64 = jnp.zeros((NPAD, F1), jnp.float32)
    out1p = _agg64(src3, dst3, ex1, den1p[0], den1p[1], h1, zeros64)

    h2, a2s, a2d = _tc_mid(out1p, b1.reshape(1, F1), W2,
                           att_src2.reshape(NC, 1), att_dst2.reshape(NC, 1))
    a2s = a2s.reshape(NPAD)
    a2d = a2d.reshape(NPAD)

    zeros1 = jnp.zeros((NPAD,), jnp.float32)
    den2p, ex2 = _att1(src3, dst3, a2s, a2d, zeros1)

    zeros40 = jnp.zeros((NPAD, NC), jnp.float32)
    out2p = _agg40(src3, dst3, ex2, den2p[0], den2p[1], h2, zeros40)

    out = _tc_final(out2p, b2.reshape(1, NC))
    return out[:N]
